# baseline (device time: 169990 ns/iter reference)
import jax
import jax.numpy as jnp
from jax import lax
from jax.experimental import pallas as pl
from jax.experimental.pallas import tpu as pltpu

F32 = jnp.float32
BF16 = jnp.bfloat16

B, S, D = 4, 256, 4096
H, DH, DR = 32, 128, 64
DC_SH = 128
SCALE = (DH + DR) ** -0.5


def _ring_coords(rr):
    cx = jnp.where(rr >= 2, 1, 0)
    cy = jnp.where((rr == 1) | (rr == 2), 1, 0)
    return cx, cy


def _proj_body(xb_ref, xp_ref, wdkv_ref, wuk_ref, wuv_ref, wkr_ref,
               wq_ref, wqr_ref,
               q_ref, qr_ref, kr_ref, k_ref, v_ref,
               wsend, wrecv, csend, crecv, cme, cvt,
               w_ssem, w_rsem, c_ssem, c_rsem):
    j = pl.program_id(0)
    my_x = lax.axis_index("x")
    my_y = lax.axis_index("y")
    partner = (1 - my_x, my_y)

    def _mk_w():
        return pltpu.make_async_remote_copy(
            src_ref=wsend, dst_ref=wrecv, send_sem=w_ssem, recv_sem=w_rsem,
            device_id=partner, device_id_type=pl.DeviceIdType.MESH)

    def _mk_c():
        return pltpu.make_async_remote_copy(
            src_ref=csend, dst_ref=crecv, send_sem=c_ssem, recv_sem=c_rsem,
            device_id=partner, device_id_type=pl.DeviceIdType.MESH)

    @pl.when(j == 0)
    def _():
        barrier = pltpu.get_barrier_semaphore()
        pl.semaphore_signal(barrier, inc=1, device_id=partner,
                            device_id_type=pl.DeviceIdType.MESH)
        pl.semaphore_wait(barrier, 1)
        wsend[0, :, :] = wuk_ref[...].astype(BF16)
        wsend[1, :, :] = wuv_ref[...].astype(BF16)
        _mk_w().start()
        wdkv = wdkv_ref[...].astype(BF16)
        csend[...] = jnp.dot(xp_ref[...], wdkv,
                             preferred_element_type=F32).astype(BF16)
        _mk_c().start()
        cme[...] = jnp.dot(xb_ref[...], wdkv,
                           preferred_element_type=F32).astype(BF16)

    @pl.when(j < 8)
    def _():
        cvt[j % 2] = wq_ref[...].astype(BF16)

    @pl.when((j >= 8) & (j < 16))
    def _():
        cvt[j % 2, :, :256] = wqr_ref[...].astype(BF16)

    @pl.when((j >= 1) & (j <= 8))
    def _():
        q_ref[...] = jnp.dot(xb_ref[...], cvt[(j - 1) % 2],
                             preferred_element_type=F32).astype(BF16)

    @pl.when((j >= 9) & (j <= 16))
    def _():
        qr_ref[...] = jnp.dot(xb_ref[...], cvt[(j - 1) % 2, :, :256],
                              preferred_element_type=F32).astype(BF16)

    @pl.when(j == 17)
    def _():
        kr_ref[...] = jnp.dot(xb_ref[...], wkr_ref[...].astype(BF16),
                              preferred_element_type=F32).astype(BF16)
        c_me = cme[...]
        k_loc = jnp.dot(c_me, wsend[0, :, :], preferred_element_type=F32)
        v_loc = jnp.dot(c_me, wsend[1, :, :], preferred_element_type=F32)
        _mk_c().wait()
        _mk_w().wait()
        k_ref[...] = (k_loc + jnp.dot(crecv[...], wrecv[0, :, :],
                                      preferred_element_type=F32)).astype(BF16)
        v_ref[...] = (v_loc + jnp.dot(crecv[...], wrecv[1, :, :],
                                      preferred_element_type=F32)).astype(BF16)


def _proj(xb_bf, xp_bf, Wdkv, Wuk, Wuv, Wkr, Wq, Wqr):
    c7 = lambda j: (0, jnp.clip(j, 0, 7))
    c3 = lambda j: (0, jnp.clip(j - 8, 0, 7))
    o7 = lambda j: (0, jnp.clip(j - 1, 0, 7))
    o3 = lambda j: (0, jnp.clip(j - 9, 0, 7))
    return pl.pallas_call(
        _proj_body,
        grid=(18,),
        in_specs=[
            pl.BlockSpec((S, D), lambda j: (0, 0)),
            pl.BlockSpec((S, D), lambda j: (0, 0)),
            pl.BlockSpec((D, DC_SH), lambda j: (0, 0)),
            pl.BlockSpec((DC_SH, D), lambda j: (0, 0)),
            pl.BlockSpec((DC_SH, D), lambda j: (0, 0)),
            pl.BlockSpec((D, DR), lambda j: (0, 0)),
            pl.BlockSpec((D, 512), c7),
            pl.BlockSpec((D, 256), c3),
        ],
        out_specs=[
            pl.BlockSpec((S, 512), o7),
            pl.BlockSpec((S, 256), o3),
            pl.BlockSpec((S, DR), lambda j: (0, 0)),
            pl.BlockSpec((S, D), lambda j: (0, 0)),
            pl.BlockSpec((S, D), lambda j: (0, 0)),
        ],
        out_shape=[
            jax.ShapeDtypeStruct((S, D), BF16),
            jax.ShapeDtypeStruct((S, 2048), BF16),
            jax.ShapeDtypeStruct((S, DR), BF16),
            jax.ShapeDtypeStruct((S, D), BF16),
            jax.ShapeDtypeStruct((S, D), BF16),
        ],
        scratch_shapes=[
            pltpu.VMEM((2, DC_SH, D), BF16),
            pltpu.VMEM((2, DC_SH, D), BF16),
            pltpu.VMEM((S, DC_SH), BF16),
            pltpu.VMEM((S, DC_SH), BF16),
            pltpu.VMEM((S, DC_SH), BF16),
            pltpu.VMEM((2, D, 512), BF16),
            pltpu.SemaphoreType.DMA,
            pltpu.SemaphoreType.DMA,
            pltpu.SemaphoreType.DMA,
            pltpu.SemaphoreType.DMA,
        ],
        compiler_params=pltpu.CompilerParams(
            collective_id=1, vmem_limit_bytes=100 * 1024 * 1024),
    )(xb_bf, xp_bf, Wdkv, Wuk, Wuv, Wkr, Wq, Wqr)


def _gemm_body(x_ref, w_ref, o_ref):
    o_ref[...] = jnp.dot(
        x_ref[...], w_ref[...].astype(BF16), preferred_element_type=F32
    ).astype(BF16)


def _gemm(xb_bf, W, blk=512):
    n = W.shape[1]
    return pl.pallas_call(
        _gemm_body,
        grid=(n // blk,),
        in_specs=[
            pl.BlockSpec((S, D), lambda j: (0, 0)),
            pl.BlockSpec((D, blk), lambda j: (0, j)),
        ],
        out_specs=pl.BlockSpec((S, blk), lambda j: (0, j)),
        out_shape=jax.ShapeDtypeStruct((S, n), BF16),
    )(xb_bf, W)


def _qr_kr_body(x_ref, wqr_ref, wkr_ref, qr_ref, kr_ref):
    x = x_ref[...]
    qr_ref[...] = jnp.dot(
        x, wqr_ref[...].astype(BF16), preferred_element_type=F32
    ).astype(BF16)
    kr_ref[...] = jnp.dot(
        x, wkr_ref[...].astype(BF16), preferred_element_type=F32
    ).astype(BF16)


def _qr_kr(xb_bf, Wqr, Wkr, blk=512):
    return pl.pallas_call(
        _qr_kr_body,
        grid=(Wqr.shape[1] // blk,),
        in_specs=[
            pl.BlockSpec((S, D), lambda j: (0, 0)),
            pl.BlockSpec((D, blk), lambda j: (0, j)),
            pl.BlockSpec((D, DR), lambda j: (0, 0)),
        ],
        out_specs=[
            pl.BlockSpec((S, blk), lambda j: (0, j)),
            pl.BlockSpec((S, DR), lambda j: (0, 0)),
        ],
        out_shape=[
            jax.ShapeDtypeStruct((S, Wqr.shape[1]), BF16),
            jax.ShapeDtypeStruct((S, DR), BF16),
        ],
    )(xb_bf, Wqr, Wkr)


HPB = 4


def _attn_body(q_ref, k_ref, v_ref, qr_ref, kr_ref, o_ref):
    krh = kr_ref[...]
    for i in range(HPB):
        sl = slice(i * DH, (i + 1) * DH)
        qh = q_ref[:, sl]
        kh = k_ref[:, sl]
        vh = v_ref[:, sl]
        qrh = qr_ref[i]
        s = lax.dot_general(qh, kh, (((1,), (1,)), ((), ())),
                            preferred_element_type=F32)
        s = s + lax.dot_general(qrh, krh, (((1,), (1,)), ((), ())),
                                preferred_element_type=F32)
        s = s * SCALE
        m = jnp.max(s, axis=1, keepdims=True)
        e = jnp.exp(s - m)
        p = (e / jnp.sum(e, axis=1, keepdims=True)).astype(BF16)
        o_ref[:, sl] = jnp.dot(p, vh, preferred_element_type=F32).astype(BF16)


def _attn(q, k, v, qr_hm, kr):
    return pl.pallas_call(
        _attn_body,
        grid=(H // HPB,),
        in_specs=[
            pl.BlockSpec((S, HPB * DH), lambda h: (0, h)),
            pl.BlockSpec((S, HPB * DH), lambda h: (0, h)),
            pl.BlockSpec((S, HPB * DH), lambda h: (0, h)),
            pl.BlockSpec((HPB, S, DR), lambda h: (h, 0, 0)),
            pl.BlockSpec((S, DR), lambda h: (0, 0)),
        ],
        out_specs=pl.BlockSpec((S, HPB * DH), lambda h: (0, h)),
        out_shape=jax.ShapeDtypeStruct((S, H * DH), BF16),
    )(q, k, v, qr_hm, kr)


HD = D // 2


def _wo_ag_body(o_ref, wo_ref, out_ref, me, cvt, cw, ccw,
                cw_ssem, cw_rsem, ccw_ssem, ccw_rsem):
    j = pl.program_id(0)
    my_x = lax.axis_index("x")
    my_y = lax.axis_index("y")
    r = jnp.where(my_x == 0, my_y, 3 - my_y)
    right = _ring_coords((r + 1) % 4)
    left = _ring_coords((r + 3) % 4)

    def _mk(h, comm, ssem, rsem, dev):
        return pltpu.make_async_remote_copy(
            src_ref=comm.at[h], dst_ref=comm.at[h + 1],
            send_sem=ssem.at[h], recv_sem=rsem.at[h],
            device_id=dev, device_id_type=pl.DeviceIdType.MESH)

    @pl.when(j == 0)
    def _():
        barrier = pltpu.get_barrier_semaphore()
        for nbr in (left, right):
            pl.semaphore_signal(barrier, inc=1, device_id=nbr,
                                device_id_type=pl.DeviceIdType.MESH)
        pl.semaphore_wait(barrier, 2)

    blk = 512

    @pl.when(j < 8)
    def _():
        cvt[j % 2] = wo_ref[...].astype(BF16)

    @pl.when(j >= 1)
    def _():
        me[:, pl.ds((j - 1) * blk, blk)] = jnp.dot(
            o_ref[...], cvt[(j - 1) % 2],
            preferred_element_type=F32).astype(BF16)

    @pl.when(j == 4)
    def _():
        cw[0, :, :] = me[:, :HD]
        _mk(0, cw, cw_ssem, cw_rsem, right).start()

    @pl.when(j == 8)
    def _():
        ccw[0, :, :] = me[:, HD:]
        _mk(0, ccw, ccw_ssem, ccw_rsem, left).start()
        out_ref[pl.ds(r, 1), :, :] = me[...].astype(F32).reshape(1, S, D)
        for h in range(3):
            _mk(h, cw, cw_ssem, cw_rsem, right).wait()
            if h < 2:
                _mk(h + 1, cw, cw_ssem, cw_rsem, right).start()
            o_cw = (r - h - 1) % 4
            out_ref[pl.ds(o_cw, 1), :, :HD] = (
                cw[h + 1, :, :].astype(F32).reshape(1, S, HD))
            _mk(h, ccw, ccw_ssem, ccw_rsem, left).wait()
            if h < 2:
                _mk(h + 1, ccw, ccw_ssem, ccw_rsem, left).start()
            o_ccw = (r + h + 1) % 4
            out_ref[pl.ds(o_ccw, 1), :, HD:] = (
                ccw[h + 1, :, :].astype(F32).reshape(1, S, HD))


def _wo_ag(o, Wo, blk=512):
    return pl.pallas_call(
        _wo_ag_body,
        grid=(9,),
        in_specs=[
            pl.BlockSpec((S, D), lambda j: (0, 0)),
            pl.BlockSpec((D, blk), lambda j: (0, jnp.clip(j, 0, 7))),
        ],
        out_specs=pl.BlockSpec((B, S, D), lambda j: (0, 0, 0)),
        out_shape=jax.ShapeDtypeStruct((B, S, D), F32),
        scratch_shapes=[
            pltpu.VMEM((S, D), BF16),
            pltpu.VMEM((2, D, 512), BF16),
            pltpu.VMEM((4, S, HD), BF16),
            pltpu.VMEM((4, S, HD), BF16),
            pltpu.SemaphoreType.DMA((3,)),
            pltpu.SemaphoreType.DMA((3,)),
            pltpu.SemaphoreType.DMA((3,)),
            pltpu.SemaphoreType.DMA((3,)),
        ],
        compiler_params=pltpu.CompilerParams(
            collective_id=0, vmem_limit_bytes=100 * 1024 * 1024),
    )(o, Wo)


def kernel(x, Wdkv, Wuk, Wuv, Wq, Wqr, Wkr, Wo):
    my_x = lax.axis_index("x")
    my_y = lax.axis_index("y")
    r = jnp.where(my_x == 0, my_y, 3 - my_y)
    pr = jnp.where(my_x == 1, my_y, 3 - my_y)

    xb_bf = lax.dynamic_index_in_dim(x, r, 0, keepdims=False).astype(BF16)
    xp_bf = lax.dynamic_index_in_dim(x, pr, 0, keepdims=False).astype(BF16)

    q_flat, qr_flat, kr, k_flat, v_flat = _proj(
        xb_bf, xp_bf, Wdkv, Wuk, Wuv, Wkr, Wq, Wqr)

    qr_hm = qr_flat.reshape(S, H, DR).transpose(1, 0, 2)
    o = _attn(q_flat, k_flat, v_flat, qr_hm, kr)

    return _wo_ag(o, Wo)


# device time: 164212 ns/iter; 1.0352x vs baseline; 1.0352x over previous
import jax
import jax.numpy as jnp
from jax import lax
from jax.experimental import pallas as pl
from jax.experimental.pallas import tpu as pltpu

F32 = jnp.float32
BF16 = jnp.bfloat16

B, S, D = 4, 256, 4096
H, DH, DR = 32, 128, 64
DC_SH = 128
SCALE = (DH + DR) ** -0.5


def _ring_coords(rr):
    cx = jnp.where(rr >= 2, 1, 0)
    cy = jnp.where((rr == 1) | (rr == 2), 1, 0)
    return cx, cy


def _proj_body(xb_ref, xp_ref, wdkv_ref, wuk_ref, wuv_ref, wkr_ref,
               wq_ref, wqr_ref,
               q_ref, qr_ref, kr_ref, k_ref, v_ref,
               wsend, wrecv, csend, crecv, cme,
               w_ssem, w_rsem, c_ssem, c_rsem):
    j = pl.program_id(0)
    my_x = lax.axis_index("x")
    my_y = lax.axis_index("y")
    partner = (1 - my_x, my_y)

    def _mk_w():
        return pltpu.make_async_remote_copy(
            src_ref=wsend, dst_ref=wrecv, send_sem=w_ssem, recv_sem=w_rsem,
            device_id=partner, device_id_type=pl.DeviceIdType.MESH)

    def _mk_c():
        return pltpu.make_async_remote_copy(
            src_ref=csend, dst_ref=crecv, send_sem=c_ssem, recv_sem=c_rsem,
            device_id=partner, device_id_type=pl.DeviceIdType.MESH)

    @pl.when(j == 0)
    def _():
        barrier = pltpu.get_barrier_semaphore()
        pl.semaphore_signal(barrier, inc=1, device_id=partner,
                            device_id_type=pl.DeviceIdType.MESH)
        pl.semaphore_wait(barrier, 1)
        wsend[0, :, :] = wuk_ref[...].astype(BF16)
        wsend[1, :, :] = wuv_ref[...].astype(BF16)
        _mk_w().start()
        wdkv = wdkv_ref[...].astype(BF16)
        csend[...] = jnp.dot(xp_ref[...], wdkv,
                             preferred_element_type=F32).astype(BF16)
        _mk_c().start()
        cme[...] = jnp.dot(xb_ref[...], wdkv,
                           preferred_element_type=F32).astype(BF16)

    @pl.when(j < 8)
    def _():
        q_ref[...] = jnp.dot(xb_ref[...], wq_ref[...].astype(BF16),
                             preferred_element_type=F32).astype(BF16)

    @pl.when((j >= 8) & (j < 12))
    def _():
        qr_ref[...] = jnp.dot(xb_ref[...], wqr_ref[...].astype(BF16),
                              preferred_element_type=F32).astype(BF16)

    @pl.when(j == 12)
    def _():
        kr_ref[...] = jnp.dot(xb_ref[...], wkr_ref[...].astype(BF16),
                              preferred_element_type=F32).astype(BF16)
        c_me = cme[...]
        k_loc = jnp.dot(c_me, wsend[0, :, :], preferred_element_type=F32)
        v_loc = jnp.dot(c_me, wsend[1, :, :], preferred_element_type=F32)
        _mk_c().wait()
        _mk_w().wait()
        k_ref[...] = (k_loc + jnp.dot(crecv[...], wrecv[0, :, :],
                                      preferred_element_type=F32)).astype(BF16)
        v_ref[...] = (v_loc + jnp.dot(crecv[...], wrecv[1, :, :],
                                      preferred_element_type=F32)).astype(BF16)


def _proj(xb_bf, xp_bf, Wdkv, Wuk, Wuv, Wkr, Wq, Wqr):
    c7 = lambda j: (0, jnp.clip(j, 0, 7))
    c3 = lambda j: (0, jnp.clip(j - 8, 0, 3))
    return pl.pallas_call(
        _proj_body,
        grid=(13,),
        in_specs=[
            pl.BlockSpec((S, D), lambda j: (0, 0)),
            pl.BlockSpec((S, D), lambda j: (0, 0)),
            pl.BlockSpec((D, DC_SH), lambda j: (0, 0)),
            pl.BlockSpec((DC_SH, D), lambda j: (0, 0)),
            pl.BlockSpec((DC_SH, D), lambda j: (0, 0)),
            pl.BlockSpec((D, DR), lambda j: (0, 0)),
            pl.BlockSpec((D, 512), c7),
            pl.BlockSpec((D, 512), c3),
        ],
        out_specs=[
            pl.BlockSpec((S, 512), c7),
            pl.BlockSpec((S, 512), c3),
            pl.BlockSpec((S, DR), lambda j: (0, 0)),
            pl.BlockSpec((S, D), lambda j: (0, 0)),
            pl.BlockSpec((S, D), lambda j: (0, 0)),
        ],
        out_shape=[
            jax.ShapeDtypeStruct((S, D), BF16),
            jax.ShapeDtypeStruct((S, 2048), BF16),
            jax.ShapeDtypeStruct((S, DR), BF16),
            jax.ShapeDtypeStruct((S, D), BF16),
            jax.ShapeDtypeStruct((S, D), BF16),
        ],
        scratch_shapes=[
            pltpu.VMEM((2, DC_SH, D), BF16),
            pltpu.VMEM((2, DC_SH, D), BF16),
            pltpu.VMEM((S, DC_SH), BF16),
            pltpu.VMEM((S, DC_SH), BF16),
            pltpu.VMEM((S, DC_SH), BF16),
            pltpu.SemaphoreType.DMA,
            pltpu.SemaphoreType.DMA,
            pltpu.SemaphoreType.DMA,
            pltpu.SemaphoreType.DMA,
        ],
        compiler_params=pltpu.CompilerParams(
            collective_id=1, vmem_limit_bytes=100 * 1024 * 1024),
    )(xb_bf, xp_bf, Wdkv, Wuk, Wuv, Wkr, Wq, Wqr)


def _gemm_body(x_ref, w_ref, o_ref):
    o_ref[...] = jnp.dot(
        x_ref[...], w_ref[...].astype(BF16), preferred_element_type=F32
    ).astype(BF16)


def _gemm(xb_bf, W, blk=512):
    n = W.shape[1]
    return pl.pallas_call(
        _gemm_body,
        grid=(n // blk,),
        in_specs=[
            pl.BlockSpec((S, D), lambda j: (0, 0)),
            pl.BlockSpec((D, blk), lambda j: (0, j)),
        ],
        out_specs=pl.BlockSpec((S, blk), lambda j: (0, j)),
        out_shape=jax.ShapeDtypeStruct((S, n), BF16),
    )(xb_bf, W)


def _qr_kr_body(x_ref, wqr_ref, wkr_ref, qr_ref, kr_ref):
    x = x_ref[...]
    qr_ref[...] = jnp.dot(
        x, wqr_ref[...].astype(BF16), preferred_element_type=F32
    ).astype(BF16)
    kr_ref[...] = jnp.dot(
        x, wkr_ref[...].astype(BF16), preferred_element_type=F32
    ).astype(BF16)


def _qr_kr(xb_bf, Wqr, Wkr, blk=512):
    return pl.pallas_call(
        _qr_kr_body,
        grid=(Wqr.shape[1] // blk,),
        in_specs=[
            pl.BlockSpec((S, D), lambda j: (0, 0)),
            pl.BlockSpec((D, blk), lambda j: (0, j)),
            pl.BlockSpec((D, DR), lambda j: (0, 0)),
        ],
        out_specs=[
            pl.BlockSpec((S, blk), lambda j: (0, j)),
            pl.BlockSpec((S, DR), lambda j: (0, 0)),
        ],
        out_shape=[
            jax.ShapeDtypeStruct((S, Wqr.shape[1]), BF16),
            jax.ShapeDtypeStruct((S, DR), BF16),
        ],
    )(xb_bf, Wqr, Wkr)


HPB = 4


def _attn_body(q_ref, k_ref, v_ref, qr_ref, kr_ref, o_ref):
    krh = kr_ref[...]
    for i in range(HPB):
        sl = slice(i * DH, (i + 1) * DH)
        qh = q_ref[:, sl]
        kh = k_ref[:, sl]
        vh = v_ref[:, sl]
        qrh = qr_ref[i]
        s = lax.dot_general(qh, kh, (((1,), (1,)), ((), ())),
                            preferred_element_type=F32)
        s = s + lax.dot_general(qrh, krh, (((1,), (1,)), ((), ())),
                                preferred_element_type=F32)
        s = s * SCALE
        m = jnp.max(s, axis=1, keepdims=True)
        e = jnp.exp(s - m)
        p = (e / jnp.sum(e, axis=1, keepdims=True)).astype(BF16)
        o_ref[:, sl] = jnp.dot(p, vh, preferred_element_type=F32).astype(BF16)


def _attn(q, k, v, qr_hm, kr):
    return pl.pallas_call(
        _attn_body,
        grid=(H // HPB,),
        in_specs=[
            pl.BlockSpec((S, HPB * DH), lambda h: (0, h)),
            pl.BlockSpec((S, HPB * DH), lambda h: (0, h)),
            pl.BlockSpec((S, HPB * DH), lambda h: (0, h)),
            pl.BlockSpec((HPB, S, DR), lambda h: (h, 0, 0)),
            pl.BlockSpec((S, DR), lambda h: (0, 0)),
        ],
        out_specs=pl.BlockSpec((S, HPB * DH), lambda h: (0, h)),
        out_shape=jax.ShapeDtypeStruct((S, H * DH), BF16),
    )(q, k, v, qr_hm, kr)


HD = D // 2


def _wo_ag_body(o_ref, wo_ref, out_ref, me, cw, ccw,
                cw_ssem, cw_rsem, ccw_ssem, ccw_rsem):
    j = pl.program_id(0)
    my_x = lax.axis_index("x")
    my_y = lax.axis_index("y")
    r = jnp.where(my_x == 0, my_y, 3 - my_y)
    right = _ring_coords((r + 1) % 4)
    left = _ring_coords((r + 3) % 4)

    def _mk(h, comm, ssem, rsem, dev):
        return pltpu.make_async_remote_copy(
            src_ref=comm.at[h], dst_ref=comm.at[h + 1],
            send_sem=ssem.at[h], recv_sem=rsem.at[h],
            device_id=dev, device_id_type=pl.DeviceIdType.MESH)

    @pl.when(j == 0)
    def _():
        barrier = pltpu.get_barrier_semaphore()
        for nbr in (left, right):
            pl.semaphore_signal(barrier, inc=1, device_id=nbr,
                                device_id_type=pl.DeviceIdType.MESH)
        pl.semaphore_wait(barrier, 2)

    blk = 512
    me[:, pl.ds(j * blk, blk)] = jnp.dot(
        o_ref[...], wo_ref[...].astype(BF16),
        preferred_element_type=F32).astype(BF16)

    @pl.when(j == 3)
    def _():
        cw[0, :, :] = me[:, :HD]
        _mk(0, cw, cw_ssem, cw_rsem, right).start()

    @pl.when(j == 7)
    def _():
        ccw[0, :, :] = me[:, HD:]
        _mk(0, ccw, ccw_ssem, ccw_rsem, left).start()
        out_ref[pl.ds(r, 1), :, :] = me[...].astype(F32).reshape(1, S, D)
        for h in range(3):
            _mk(h, cw, cw_ssem, cw_rsem, right).wait()
            if h < 2:
                _mk(h + 1, cw, cw_ssem, cw_rsem, right).start()
            o_cw = (r - h - 1) % 4
            out_ref[pl.ds(o_cw, 1), :, :HD] = (
                cw[h + 1, :, :].astype(F32).reshape(1, S, HD))
            _mk(h, ccw, ccw_ssem, ccw_rsem, left).wait()
            if h < 2:
                _mk(h + 1, ccw, ccw_ssem, ccw_rsem, left).start()
            o_ccw = (r + h + 1) % 4
            out_ref[pl.ds(o_ccw, 1), :, HD:] = (
                ccw[h + 1, :, :].astype(F32).reshape(1, S, HD))


def _wo_ag(o, Wo, blk=512):
    return pl.pallas_call(
        _wo_ag_body,
        grid=(8,),
        in_specs=[
            pl.BlockSpec((S, D), lambda j: (0, 0)),
            pl.BlockSpec((D, blk), lambda j: (0, j)),
        ],
        out_specs=pl.BlockSpec((B, S, D), lambda j: (0, 0, 0)),
        out_shape=jax.ShapeDtypeStruct((B, S, D), F32),
        scratch_shapes=[
            pltpu.VMEM((S, D), BF16),
            pltpu.VMEM((4, S, HD), BF16),
            pltpu.VMEM((4, S, HD), BF16),
            pltpu.SemaphoreType.DMA((3,)),
            pltpu.SemaphoreType.DMA((3,)),
            pltpu.SemaphoreType.DMA((3,)),
            pltpu.SemaphoreType.DMA((3,)),
        ],
        compiler_params=pltpu.CompilerParams(
            collective_id=0, vmem_limit_bytes=100 * 1024 * 1024),
    )(o, Wo)


def kernel(x, Wdkv, Wuk, Wuv, Wq, Wqr, Wkr, Wo):
    my_x = lax.axis_index("x")
    my_y = lax.axis_index("y")
    r = jnp.where(my_x == 0, my_y, 3 - my_y)
    pr = jnp.where(my_x == 1, my_y, 3 - my_y)

    xb_bf = lax.dynamic_index_in_dim(x, r, 0, keepdims=False).astype(BF16)
    xp_bf = lax.dynamic_index_in_dim(x, pr, 0, keepdims=False).astype(BF16)

    q_flat, qr_flat, kr, k_flat, v_flat = _proj(
        xb_bf, xp_bf, Wdkv, Wuk, Wuv, Wkr, Wq, Wqr)

    qr_hm = qr_flat.reshape(S, H, DR).transpose(1, 0, 2)
    o = _attn(q_flat, k_flat, v_flat, qr_hm, kr)

    return _wo_ag(o, Wo)


# device time: 157278 ns/iter; 1.0808x vs baseline; 1.0441x over previous
import jax
import jax.numpy as jnp
from jax import lax
from jax.experimental import pallas as pl
from jax.experimental.pallas import tpu as pltpu

F32 = jnp.float32
BF16 = jnp.bfloat16

B, S, D = 4, 256, 4096
H, DH, DR = 32, 128, 64
DC_SH = 128
SCALE = (DH + DR) ** -0.5


def _ring_coords(rr):
    cx = jnp.where(rr >= 2, 1, 0)
    cy = jnp.where((rr == 1) | (rr == 2), 1, 0)
    return cx, cy


def _proj_body(xb_ref, xp_ref, wdkv_ref, wuk_ref, wuv_ref, wkr_ref,
               wq_ref, wqr_ref,
               o_ref,
               qs, qrs, ks, vs,
               wsend, wrecv, csend, crecv, cme,
               w_ssem, w_rsem, c_ssem, c_rsem):
    j = pl.program_id(0)
    my_x = lax.axis_index("x")
    my_y = lax.axis_index("y")
    partner = (1 - my_x, my_y)

    def _mk_w():
        return pltpu.make_async_remote_copy(
            src_ref=wsend, dst_ref=wrecv, send_sem=w_ssem, recv_sem=w_rsem,
            device_id=partner, device_id_type=pl.DeviceIdType.MESH)

    def _mk_c():
        return pltpu.make_async_remote_copy(
            src_ref=csend, dst_ref=crecv, send_sem=c_ssem, recv_sem=c_rsem,
            device_id=partner, device_id_type=pl.DeviceIdType.MESH)

    @pl.when(j == 0)
    def _():
        barrier = pltpu.get_barrier_semaphore()
        pl.semaphore_signal(barrier, inc=1, device_id=partner,
                            device_id_type=pl.DeviceIdType.MESH)
        pl.semaphore_wait(barrier, 1)
        wsend[0, :, :] = wuk_ref[...].astype(BF16)
        wsend[1, :, :] = wuv_ref[...].astype(BF16)
        _mk_w().start()
        wdkv = wdkv_ref[...].astype(BF16)
        csend[...] = jnp.dot(xp_ref[...], wdkv,
                             preferred_element_type=F32).astype(BF16)
        _mk_c().start()
        cme[...] = jnp.dot(xb_ref[...], wdkv,
                           preferred_element_type=F32).astype(BF16)

    @pl.when(j < 8)
    def _():
        qs[:, pl.ds(j * 512, 512)] = jnp.dot(
            xb_ref[...], wq_ref[...].astype(BF16),
            preferred_element_type=F32).astype(BF16)

    @pl.when((j >= 8) & (j < 12))
    def _():
        qrs[:, pl.ds((j - 8) * 512, 512)] = jnp.dot(
            xb_ref[...], wqr_ref[...].astype(BF16),
            preferred_element_type=F32).astype(BF16)

    @pl.when(j == 12)
    def _():
        krh = jnp.dot(xb_ref[...], wkr_ref[...].astype(BF16),
                      preferred_element_type=F32).astype(BF16)
        c_me = cme[...]
        k_loc = jnp.dot(c_me, wsend[0, :, :], preferred_element_type=F32)
        v_loc = jnp.dot(c_me, wsend[1, :, :], preferred_element_type=F32)
        _mk_c().wait()
        _mk_w().wait()
        ks[...] = (k_loc + jnp.dot(crecv[...], wrecv[0, :, :],
                                   preferred_element_type=F32)).astype(BF16)
        vs[...] = (v_loc + jnp.dot(crecv[...], wrecv[1, :, :],
                                   preferred_element_type=F32)).astype(BF16)
        for hp in range(H // 2):
            qr2 = qrs[:, hp * 128:(hp + 1) * 128]
            for i in range(2):
                h = 2 * hp + i
                sl = slice(h * DH, (h + 1) * DH)
                qh = qs[:, sl]
                kh = ks[:, sl]
                vh = vs[:, sl]
                qrh = qr2[:, i * DR:(i + 1) * DR]
                s = lax.dot_general(qh, kh, (((1,), (1,)), ((), ())),
                                    preferred_element_type=F32)
                s = s + lax.dot_general(qrh, krh, (((1,), (1,)), ((), ())),
                                        preferred_element_type=F32)
                s = s * SCALE
                m = jnp.max(s, axis=1, keepdims=True)
                e = jnp.exp(s - m)
                p = (e / jnp.sum(e, axis=1, keepdims=True)).astype(BF16)
                o_ref[:, sl] = jnp.dot(p, vh,
                                       preferred_element_type=F32).astype(BF16)


def _proj(xb_bf, xp_bf, Wdkv, Wuk, Wuv, Wkr, Wq, Wqr):
    c7 = lambda j: (0, jnp.clip(j, 0, 7))
    c3 = lambda j: (0, jnp.clip(j - 8, 0, 3))
    return pl.pallas_call(
        _proj_body,
        grid=(13,),
        in_specs=[
            pl.BlockSpec((S, D), lambda j: (0, 0)),
            pl.BlockSpec((S, D), lambda j: (0, 0)),
            pl.BlockSpec((D, DC_SH), lambda j: (0, 0)),
            pl.BlockSpec((DC_SH, D), lambda j: (0, 0)),
            pl.BlockSpec((DC_SH, D), lambda j: (0, 0)),
            pl.BlockSpec((D, DR), lambda j: (0, 0)),
            pl.BlockSpec((D, 512), c7),
            pl.BlockSpec((D, 512), c3),
        ],
        out_specs=pl.BlockSpec((S, D), lambda j: (0, 0)),
        out_shape=jax.ShapeDtypeStruct((S, D), BF16),
        scratch_shapes=[
            pltpu.VMEM((S, D), BF16),
            pltpu.VMEM((S, 2048), BF16),
            pltpu.VMEM((S, D), BF16),
            pltpu.VMEM((S, D), BF16),
            pltpu.VMEM((2, DC_SH, D), BF16),
            pltpu.VMEM((2, DC_SH, D), BF16),
            pltpu.VMEM((S, DC_SH), BF16),
            pltpu.VMEM((S, DC_SH), BF16),
            pltpu.VMEM((S, DC_SH), BF16),
            pltpu.SemaphoreType.DMA,
            pltpu.SemaphoreType.DMA,
            pltpu.SemaphoreType.DMA,
            pltpu.SemaphoreType.DMA,
        ],
        compiler_params=pltpu.CompilerParams(
            collective_id=1, vmem_limit_bytes=100 * 1024 * 1024),
    )(xb_bf, xp_bf, Wdkv, Wuk, Wuv, Wkr, Wq, Wqr)


def _gemm_body(x_ref, w_ref, o_ref):
    o_ref[...] = jnp.dot(
        x_ref[...], w_ref[...].astype(BF16), preferred_element_type=F32
    ).astype(BF16)


def _gemm(xb_bf, W, blk=512):
    n = W.shape[1]
    return pl.pallas_call(
        _gemm_body,
        grid=(n // blk,),
        in_specs=[
            pl.BlockSpec((S, D), lambda j: (0, 0)),
            pl.BlockSpec((D, blk), lambda j: (0, j)),
        ],
        out_specs=pl.BlockSpec((S, blk), lambda j: (0, j)),
        out_shape=jax.ShapeDtypeStruct((S, n), BF16),
    )(xb_bf, W)


def _qr_kr_body(x_ref, wqr_ref, wkr_ref, qr_ref, kr_ref):
    x = x_ref[...]
    qr_ref[...] = jnp.dot(
        x, wqr_ref[...].astype(BF16), preferred_element_type=F32
    ).astype(BF16)
    kr_ref[...] = jnp.dot(
        x, wkr_ref[...].astype(BF16), preferred_element_type=F32
    ).astype(BF16)


def _qr_kr(xb_bf, Wqr, Wkr, blk=512):
    return pl.pallas_call(
        _qr_kr_body,
        grid=(Wqr.shape[1] // blk,),
        in_specs=[
            pl.BlockSpec((S, D), lambda j: (0, 0)),
            pl.BlockSpec((D, blk), lambda j: (0, j)),
            pl.BlockSpec((D, DR), lambda j: (0, 0)),
        ],
        out_specs=[
            pl.BlockSpec((S, blk), lambda j: (0, j)),
            pl.BlockSpec((S, DR), lambda j: (0, 0)),
        ],
        out_shape=[
            jax.ShapeDtypeStruct((S, Wqr.shape[1]), BF16),
            jax.ShapeDtypeStruct((S, DR), BF16),
        ],
    )(xb_bf, Wqr, Wkr)


HPB = 4


def _attn_body(q_ref, k_ref, v_ref, qr_ref, kr_ref, o_ref):
    krh = kr_ref[...]
    for i in range(HPB):
        sl = slice(i * DH, (i + 1) * DH)
        qh = q_ref[:, sl]
        kh = k_ref[:, sl]
        vh = v_ref[:, sl]
        qrh = qr_ref[i]
        s = lax.dot_general(qh, kh, (((1,), (1,)), ((), ())),
                            preferred_element_type=F32)
        s = s + lax.dot_general(qrh, krh, (((1,), (1,)), ((), ())),
                                preferred_element_type=F32)
        s = s * SCALE
        m = jnp.max(s, axis=1, keepdims=True)
        e = jnp.exp(s - m)
        p = (e / jnp.sum(e, axis=1, keepdims=True)).astype(BF16)
        o_ref[:, sl] = jnp.dot(p, vh, preferred_element_type=F32).astype(BF16)


def _attn(q, k, v, qr_hm, kr):
    return pl.pallas_call(
        _attn_body,
        grid=(H // HPB,),
        in_specs=[
            pl.BlockSpec((S, HPB * DH), lambda h: (0, h)),
            pl.BlockSpec((S, HPB * DH), lambda h: (0, h)),
            pl.BlockSpec((S, HPB * DH), lambda h: (0, h)),
            pl.BlockSpec((HPB, S, DR), lambda h: (h, 0, 0)),
            pl.BlockSpec((S, DR), lambda h: (0, 0)),
        ],
        out_specs=pl.BlockSpec((S, HPB * DH), lambda h: (0, h)),
        out_shape=jax.ShapeDtypeStruct((S, H * DH), BF16),
    )(q, k, v, qr_hm, kr)


HD = D // 2


def _wo_ag_body(o_ref, wo_ref, out_ref, me, cw, ccw,
                cw_ssem, cw_rsem, ccw_ssem, ccw_rsem):
    j = pl.program_id(0)
    my_x = lax.axis_index("x")
    my_y = lax.axis_index("y")
    r = jnp.where(my_x == 0, my_y, 3 - my_y)
    right = _ring_coords((r + 1) % 4)
    left = _ring_coords((r + 3) % 4)

    def _mk(h, comm, ssem, rsem, dev):
        return pltpu.make_async_remote_copy(
            src_ref=comm.at[h], dst_ref=comm.at[h + 1],
            send_sem=ssem.at[h], recv_sem=rsem.at[h],
            device_id=dev, device_id_type=pl.DeviceIdType.MESH)

    @pl.when(j == 0)
    def _():
        barrier = pltpu.get_barrier_semaphore()
        for nbr in (left, right):
            pl.semaphore_signal(barrier, inc=1, device_id=nbr,
                                device_id_type=pl.DeviceIdType.MESH)
        pl.semaphore_wait(barrier, 2)

    blk = 512
    me[:, pl.ds(j * blk, blk)] = jnp.dot(
        o_ref[...], wo_ref[...].astype(BF16),
        preferred_element_type=F32).astype(BF16)

    @pl.when(j == 3)
    def _():
        cw[0, :, :] = me[:, :HD]
        _mk(0, cw, cw_ssem, cw_rsem, right).start()

    @pl.when(j == 7)
    def _():
        ccw[0, :, :] = me[:, HD:]
        _mk(0, ccw, ccw_ssem, ccw_rsem, left).start()
        out_ref[pl.ds(r, 1), :, :] = me[...].astype(F32).reshape(1, S, D)
        for h in range(3):
            _mk(h, cw, cw_ssem, cw_rsem, right).wait()
            if h < 2:
                _mk(h + 1, cw, cw_ssem, cw_rsem, right).start()
            o_cw = (r - h - 1) % 4
            out_ref[pl.ds(o_cw, 1), :, :HD] = (
                cw[h + 1, :, :].astype(F32).reshape(1, S, HD))
            _mk(h, ccw, ccw_ssem, ccw_rsem, left).wait()
            if h < 2:
                _mk(h + 1, ccw, ccw_ssem, ccw_rsem, left).start()
            o_ccw = (r + h + 1) % 4
            out_ref[pl.ds(o_ccw, 1), :, HD:] = (
                ccw[h + 1, :, :].astype(F32).reshape(1, S, HD))


def _wo_ag(o, Wo, blk=512):
    return pl.pallas_call(
        _wo_ag_body,
        grid=(8,),
        in_specs=[
            pl.BlockSpec((S, D), lambda j: (0, 0)),
            pl.BlockSpec((D, blk), lambda j: (0, j)),
        ],
        out_specs=pl.BlockSpec((B, S, D), lambda j: (0, 0, 0)),
        out_shape=jax.ShapeDtypeStruct((B, S, D), F32),
        scratch_shapes=[
            pltpu.VMEM((S, D), BF16),
            pltpu.VMEM((4, S, HD), BF16),
            pltpu.VMEM((4, S, HD), BF16),
            pltpu.SemaphoreType.DMA((3,)),
            pltpu.SemaphoreType.DMA((3,)),
            pltpu.SemaphoreType.DMA((3,)),
            pltpu.SemaphoreType.DMA((3,)),
        ],
        compiler_params=pltpu.CompilerParams(
            collective_id=0, vmem_limit_bytes=100 * 1024 * 1024),
    )(o, Wo)


def kernel(x, Wdkv, Wuk, Wuv, Wq, Wqr, Wkr, Wo):
    my_x = lax.axis_index("x")
    my_y = lax.axis_index("y")
    r = jnp.where(my_x == 0, my_y, 3 - my_y)
    pr = jnp.where(my_x == 1, my_y, 3 - my_y)

    xb_bf = lax.dynamic_index_in_dim(x, r, 0, keepdims=False).astype(BF16)
    xp_bf = lax.dynamic_index_in_dim(x, pr, 0, keepdims=False).astype(BF16)

    o = _proj(xb_bf, xp_bf, Wdkv, Wuk, Wuv, Wkr, Wq, Wqr)
    return _wo_ag(o, Wo)
